# Initial kernel scaffold; baseline (speedup 1.0000x reference)
import jax
import jax.numpy as jnp
from jax.experimental import pallas as pl


def kernel(X, M_0, Wk, bk, Wv, bv, Wg, bg, Wq, bq):
    raise NotImplementedError("write your pallas kernel here")


# speedup vs baseline: 16.0378x; 16.0378x over previous
import functools

import jax
import jax.numpy as jnp
from jax import lax
from jax.experimental import pallas as pl
from jax.experimental.pallas import tpu as pltpu
from jax.experimental.pallas import tpu_sc as plsc

SEQ, B, D, H, N, K = 32, 32, 1024, 128, 16, 2
NSLOT = N + 1
L = 16
HC = H // L

COL_G = 0
COL_Q = COL_G + N
COL_K = COL_Q + H
COL_V = COL_K + H * NSLOT
NOUT_RAW = COL_V + H * NSLOT
NOUT = 4608
BLK_N = 512


def _gemm_body(x_ref, w_ref, b_ref, o_ref):
    o_ref[...] = (
        jnp.dot(x_ref[...], w_ref[...], preferred_element_type=jnp.float32)
        + b_ref[...]
    )


def _tc_gemm(x, w, b):
    m = x.shape[0]
    return pl.pallas_call(
        _gemm_body,
        grid=(NOUT // BLK_N,),
        in_specs=[
            pl.BlockSpec((m, D), lambda j: (0, 0)),
            pl.BlockSpec((D, BLK_N), lambda j: (0, j)),
            pl.BlockSpec((1, BLK_N), lambda j: (0, j)),
        ],
        out_specs=pl.BlockSpec((m, BLK_N), lambda j: (0, j)),
        out_shape=jax.ShapeDtypeStruct((m, NOUT), jnp.float32),
    )(x, w, b)


def _scalar(v):
    return v if getattr(v, "ndim", 0) == 0 else v[0]


def _make_sc_kernel():
    info = plsc.get_sparse_core_info()
    nc = info.num_cores
    mesh = plsc.VectorSubcoreMesh(core_axis_name="c", subcore_axis_name="s")

    @functools.partial(
        pl.kernel,
        mesh=mesh,
        compiler_params=pltpu.CompilerParams(needs_layout_passes=False),
        out_type=[
            jax.ShapeDtypeStruct((SEQ * B * H,), jnp.float32),
            jax.ShapeDtypeStruct((B * NSLOT, H, H), jnp.float32),
        ],
        scratch_types=[
            pltpu.VMEM((H, H), jnp.float32),
            pltpu.VMEM((H, H), jnp.float32),
            pltpu.VMEM((H, H), jnp.float32),
            pltpu.VMEM((L,), jnp.float32),
            pltpu.VMEM((H,), jnp.float32),
            pltpu.VMEM((H,), jnp.float32),
            pltpu.VMEM((H,), jnp.float32),
            pltpu.VMEM((H,), jnp.float32),
            pltpu.VMEM((H,), jnp.float32),
            pltpu.VMEM((H,), jnp.float32),
            pltpu.VMEM((H,), jnp.float32),
            pltpu.VMEM((H,), jnp.float32),
        ],
    )
    def sc_fn(y_hbm, m0_hbm, o_hbm, m_hbm,
              blkA, blkB, blkC, lg, qv, kA, vA, kB, vB, kC, vC, ov):
        b = lax.axis_index("s") * nc + lax.axis_index("c")

        def init_slot(s, carry):
            pltpu.sync_copy(m0_hbm.at[b * NSLOT + s], blkA)
            pltpu.sync_copy(blkA, m_hbm.at[b * NSLOT + s])
            return carry

        lax.fori_loop(0, NSLOT, init_slot, 0)

        def rank1_and_dot(blk, kv, vv, upd_w, acc_scale):
            vvcs = [vv[pl.ds(c * L, L)] for c in range(HC)]

            def rcloop(rc, accs):
                accs = list(accs)
                base = rc * L
                k16 = kv[pl.ds(base, L)] * upd_w
                q16 = qv[pl.ds(base, L)]
                for rl in range(L):
                    kr = k16[rl]
                    qr = q16[rl]
                    r = base + rl
                    for c in range(HC):
                        sl = pl.ds(c * L, L)
                        mrow = blk[r, sl] + kr * vvcs[c]
                        blk[r, sl] = mrow
                        accs[c] = accs[c] + qr * mrow
                return tuple(accs)

            accs = lax.fori_loop(
                0, H // L, rcloop,
                tuple(jnp.zeros((L,), jnp.float32) for _ in range(HC)),
            )
            for c in range(HC):
                sl = pl.ds(c * L, L)
                ov[sl] = ov[sl] + acc_scale * accs[c]

        def step(t, carry):
            row = t * B + b
            pltpu.sync_copy(y_hbm.at[pl.ds(row * NOUT + COL_G, L)], lg)
            l = lg[...]
            iot = lax.iota(jnp.int32, 16)
            skeys, svals = plsc.sort_key_val(l, iot, descending=True)
            idx0 = svals[0]
            idx1 = svals[1]
            b0 = jnp.full((L,), skeys[0], dtype=jnp.float32)
            b1 = jnp.full((L,), skeys[1], dtype=jnp.float32)
            ev = jnp.exp(b1 - b0)
            one = jnp.full((L,), 1.0, dtype=jnp.float32)
            g0 = one / (one + ev)
            g1 = ev * g0

            i0z = jnp.where(jnp.full((L,), idx0) == 0, 1.0, 0.0)
            i1z = jnp.where(jnp.full((L,), idx1) == 0, 1.0, 0.0)
            c0 = one + i0z + i1z
            wA = one + g0 * i0z + g1 * i1z

            pltpu.sync_copy(y_hbm.at[pl.ds(row * NOUT + COL_Q, H)], qv)
            pltpu.sync_copy(y_hbm.at[pl.ds(row * NOUT + COL_K, H)], kA)
            pltpu.sync_copy(y_hbm.at[pl.ds(row * NOUT + COL_V, H)], vA)
            pltpu.sync_copy(m_hbm.at[b * NSLOT], blkA)

            for c in range(HC):
                ov[pl.ds(c * L, L)] = jnp.zeros((L,), jnp.float32)

            rank1_and_dot(blkA, kA, vA, c0, wA)
            pltpu.sync_copy(blkA, m_hbm.at[b * NSLOT])

            @pl.when(idx0 != 0)
            def _():
                pltpu.sync_copy(y_hbm.at[pl.ds(row * NOUT + COL_K + idx0 * H, H)], kB)
                pltpu.sync_copy(y_hbm.at[pl.ds(row * NOUT + COL_V + idx0 * H, H)], vB)
                pltpu.sync_copy(m_hbm.at[b * NSLOT + idx0], blkB)
                rank1_and_dot(blkB, kB, vB, one, g0)
                pltpu.sync_copy(blkB, m_hbm.at[b * NSLOT + idx0])

            @pl.when(idx1 != 0)
            def _():
                pltpu.sync_copy(y_hbm.at[pl.ds(row * NOUT + COL_K + idx1 * H, H)], kC)
                pltpu.sync_copy(y_hbm.at[pl.ds(row * NOUT + COL_V + idx1 * H, H)], vC)
                pltpu.sync_copy(m_hbm.at[b * NSLOT + idx1], blkC)
                rank1_and_dot(blkC, kC, vC, one, g1)
                pltpu.sync_copy(blkC, m_hbm.at[b * NSLOT + idx1])

            pltpu.sync_copy(ov, o_hbm.at[pl.ds(row * H, H)])
            return carry

        lax.fori_loop(0, SEQ, step, 0)

    return sc_fn


def kernel(X, M_0, Wk, bk, Wv, bv, Wg, bg, Wq, bq):
    x_flat = X.reshape(SEQ * B, D)
    pad = jnp.zeros((D, NOUT - NOUT_RAW), jnp.float32)
    w_cat = jnp.concatenate([Wg, Wq, Wk, Wv, pad], axis=1)
    b_cat = jnp.concatenate(
        [bg, bq, bk, bv, jnp.zeros((NOUT - NOUT_RAW,), jnp.float32)]
    ).reshape(1, NOUT)
    y = _tc_gemm(x_flat, w_cat, b_cat)
    o_flat, m_flat = _make_sc_kernel()(y.reshape(-1), M_0.reshape(B * NSLOT, H, H))
    return o_flat.reshape(SEQ, B, H), m_flat.reshape(B, NSLOT, H, H)


# speedup vs baseline: 31.1575x; 1.9427x over previous
import functools

import jax
import jax.numpy as jnp
from jax import lax
from jax.experimental import pallas as pl
from jax.experimental.pallas import tpu as pltpu
from jax.experimental.pallas import tpu_sc as plsc

SEQ, B, D, H, N, K = 32, 32, 1024, 128, 16, 2
NSLOT = N + 1
L = 16
HC = H // L

COL_G = 0
COL_Q = COL_G + N
COL_K = COL_Q + H
COL_V = COL_K + H * NSLOT
NOUT_RAW = COL_V + H * NSLOT
NOUT = 4608
BLK_N = 512


def _gemm_body(x_ref, w_ref, b_ref, o_ref):
    o_ref[...] = (
        jnp.dot(x_ref[...], w_ref[...], preferred_element_type=jnp.float32)
        + b_ref[...]
    )


def _tc_gemm(x, w, b):
    m = x.shape[0]
    return pl.pallas_call(
        _gemm_body,
        grid=(NOUT // BLK_N,),
        in_specs=[
            pl.BlockSpec((m, D), lambda j: (0, 0)),
            pl.BlockSpec((D, BLK_N), lambda j: (0, j)),
            pl.BlockSpec((1, BLK_N), lambda j: (0, j)),
        ],
        out_specs=pl.BlockSpec((m, BLK_N), lambda j: (0, j)),
        out_shape=jax.ShapeDtypeStruct((m, NOUT), jnp.float32),
    )(x, w, b)


def _scalar(v):
    return v if getattr(v, "ndim", 0) == 0 else v[0]


def _make_sc_kernel():
    info = plsc.get_sparse_core_info()
    nc = info.num_cores
    mesh = plsc.VectorSubcoreMesh(core_axis_name="c", subcore_axis_name="s")

    @functools.partial(
        pl.kernel,
        mesh=mesh,
        compiler_params=pltpu.CompilerParams(needs_layout_passes=False),
        out_type=[
            jax.ShapeDtypeStruct((SEQ * B * H,), jnp.float32),
            jax.ShapeDtypeStruct((B * NSLOT, H, H), jnp.float32),
        ],
        scratch_types=[
            pltpu.VMEM((H, H), jnp.float32),
            pltpu.VMEM((H, H), jnp.float32),
            pltpu.VMEM((H, H), jnp.float32),
            pltpu.VMEM((L,), jnp.float32),
            pltpu.VMEM((H,), jnp.float32),
            pltpu.VMEM((H,), jnp.float32),
            pltpu.VMEM((H,), jnp.float32),
            pltpu.VMEM((H,), jnp.float32),
            pltpu.VMEM((H,), jnp.float32),
            pltpu.VMEM((H,), jnp.float32),
            pltpu.VMEM((H,), jnp.float32),
            pltpu.VMEM((H,), jnp.float32),
            pltpu.SemaphoreType.DMA,
            pltpu.SemaphoreType.DMA,
            pltpu.SemaphoreType.DMA,
            pltpu.SemaphoreType.DMA,
        ],
    )
    def sc_fn(y_hbm, m0_hbm, o_hbm, m_hbm,
              blkA, blkB, blkC, lg, qv, kA, vA, kB, vB, kC, vC, ov,
              semB, semC, semW, semP):
        b = lax.axis_index("s") * nc + lax.axis_index("c")

        pltpu.sync_copy(m0_hbm.at[b * NSLOT], blkA)

        def init_slot(s, carry):
            pltpu.sync_copy(m0_hbm.at[b * NSLOT + s], blkB)
            pltpu.sync_copy(blkB, m_hbm.at[b * NSLOT + s])
            return carry

        lax.fori_loop(1, NSLOT, init_slot, 0)

        row0 = b
        pltpu.async_copy(y_hbm.at[pl.ds(row0 * NOUT + COL_G, L)], lg, semP)
        pltpu.async_copy(y_hbm.at[pl.ds(row0 * NOUT + COL_Q, H)], qv, semP)
        pltpu.async_copy(y_hbm.at[pl.ds(row0 * NOUT + COL_K, H)], kA, semP)
        pltpu.async_copy(y_hbm.at[pl.ds(row0 * NOUT + COL_V, H)], vA, semP)

        def rank1_and_dot(blk, kv, vv, upd_w, acc_scale):
            vvcs = [vv[pl.ds(c * L, L)] for c in range(HC)]

            def rcloop(rc, accs):
                accs = list(accs)
                base = rc * L
                k16 = kv[pl.ds(base, L)] * upd_w
                q16 = qv[pl.ds(base, L)]
                for rl in range(L):
                    kr = k16[rl]
                    qr = q16[rl]
                    r = base + rl
                    for c in range(HC):
                        sl = pl.ds(c * L, L)
                        mrow = blk[r, sl] + kr * vvcs[c]
                        blk[r, sl] = mrow
                        accs[c] = accs[c] + qr * mrow
                return tuple(accs)

            accs = lax.fori_loop(
                0, H // L, rcloop,
                tuple(jnp.zeros((L,), jnp.float32) for _ in range(HC)),
            )
            for c in range(HC):
                sl = pl.ds(c * L, L)
                ov[sl] = ov[sl] + acc_scale * accs[c]

        def step(t, carry):
            row = t * B + b
            pltpu.make_async_copy(y_hbm.at[pl.ds(row * NOUT + COL_G, L)], lg, semP).wait()
            pltpu.make_async_copy(y_hbm.at[pl.ds(row * NOUT + COL_Q, H)], qv, semP).wait()
            pltpu.make_async_copy(y_hbm.at[pl.ds(row * NOUT + COL_K, H)], kA, semP).wait()
            pltpu.make_async_copy(y_hbm.at[pl.ds(row * NOUT + COL_V, H)], vA, semP).wait()
            l = lg[...]
            iot = lax.iota(jnp.int32, 16)
            skeys, svals = plsc.sort_key_val(l, iot, descending=True)
            idx0 = svals[0]
            idx1 = svals[1]
            b0 = jnp.full((L,), skeys[0], dtype=jnp.float32)
            b1 = jnp.full((L,), skeys[1], dtype=jnp.float32)
            ev = jnp.exp(b1 - b0)
            one = jnp.full((L,), 1.0, dtype=jnp.float32)
            g0 = one / (one + ev)
            g1 = ev * g0

            i0z = jnp.where(jnp.full((L,), idx0) == 0, 1.0, 0.0)
            i1z = jnp.where(jnp.full((L,), idx1) == 0, 1.0, 0.0)
            c0 = one + i0z + i1z
            wA = one + g0 * i0z + g1 * i1z

            @pl.when(idx0 != 0)
            def _():
                pltpu.async_copy(m_hbm.at[b * NSLOT + idx0], blkB, semB)
                pltpu.async_copy(y_hbm.at[pl.ds(row * NOUT + COL_K + idx0 * H, H)], kB, semB)
                pltpu.async_copy(y_hbm.at[pl.ds(row * NOUT + COL_V + idx0 * H, H)], vB, semB)

            @pl.when(idx1 != 0)
            def _():
                pltpu.async_copy(m_hbm.at[b * NSLOT + idx1], blkC, semC)
                pltpu.async_copy(y_hbm.at[pl.ds(row * NOUT + COL_K + idx1 * H, H)], kC, semC)
                pltpu.async_copy(y_hbm.at[pl.ds(row * NOUT + COL_V + idx1 * H, H)], vC, semC)

            for c in range(HC):
                ov[pl.ds(c * L, L)] = jnp.zeros((L,), jnp.float32)

            rank1_and_dot(blkA, kA, vA, c0, wA)

            @pl.when(idx0 != 0)
            def _():
                pltpu.make_async_copy(m_hbm.at[b * NSLOT + idx0], blkB, semB).wait()
                pltpu.make_async_copy(y_hbm.at[pl.ds(0, H)], kB, semB).wait()
                pltpu.make_async_copy(y_hbm.at[pl.ds(0, H)], vB, semB).wait()
                rank1_and_dot(blkB, kB, vB, one, g0)
                pltpu.async_copy(blkB, m_hbm.at[b * NSLOT + idx0], semW)

            @pl.when(idx1 != 0)
            def _():
                pltpu.make_async_copy(m_hbm.at[b * NSLOT + idx1], blkC, semC).wait()
                pltpu.make_async_copy(y_hbm.at[pl.ds(0, H)], kC, semC).wait()
                pltpu.make_async_copy(y_hbm.at[pl.ds(0, H)], vC, semC).wait()
                rank1_and_dot(blkC, kC, vC, one, g1)
                pltpu.async_copy(blkC, m_hbm.at[b * NSLOT + idx1], semW)

            pltpu.async_copy(ov, o_hbm.at[pl.ds(row * H, H)], semW)

            nrow = jnp.minimum(t + 1, SEQ - 1) * B + b
            pltpu.async_copy(y_hbm.at[pl.ds(nrow * NOUT + COL_G, L)], lg, semP)
            pltpu.async_copy(y_hbm.at[pl.ds(nrow * NOUT + COL_Q, H)], qv, semP)
            pltpu.async_copy(y_hbm.at[pl.ds(nrow * NOUT + COL_K, H)], kA, semP)
            pltpu.async_copy(y_hbm.at[pl.ds(nrow * NOUT + COL_V, H)], vA, semP)

            @pl.when(idx0 != 0)
            def _():
                pltpu.make_async_copy(blkB, m_hbm.at[b * NSLOT + idx0], semW).wait()

            @pl.when(idx1 != 0)
            def _():
                pltpu.make_async_copy(blkC, m_hbm.at[b * NSLOT + idx1], semW).wait()

            pltpu.make_async_copy(ov, o_hbm.at[pl.ds(row * H, H)], semW).wait()
            return carry

        lax.fori_loop(0, SEQ, step, 0)

        lrow = (SEQ - 1) * B + b
        pltpu.make_async_copy(y_hbm.at[pl.ds(lrow * NOUT + COL_G, L)], lg, semP).wait()
        pltpu.make_async_copy(y_hbm.at[pl.ds(lrow * NOUT + COL_Q, H)], qv, semP).wait()
        pltpu.make_async_copy(y_hbm.at[pl.ds(lrow * NOUT + COL_K, H)], kA, semP).wait()
        pltpu.make_async_copy(y_hbm.at[pl.ds(lrow * NOUT + COL_V, H)], vA, semP).wait()
        pltpu.sync_copy(blkA, m_hbm.at[b * NSLOT])

    return sc_fn


def kernel(X, M_0, Wk, bk, Wv, bv, Wg, bg, Wq, bq):
    x_flat = X.reshape(SEQ * B, D)
    pad = jnp.zeros((D, NOUT - NOUT_RAW), jnp.float32)
    w_cat = jnp.concatenate([Wg, Wq, Wk, Wv, pad], axis=1)
    b_cat = jnp.concatenate(
        [bg, bq, bk, bv, jnp.zeros((NOUT - NOUT_RAW,), jnp.float32)]
    ).reshape(1, NOUT)
    y = _tc_gemm(x_flat, w_cat, b_cat)
    o_flat, m_flat = _make_sc_kernel()(y.reshape(-1), M_0.reshape(B * NSLOT, H, H))
    return o_flat.reshape(SEQ, B, H), m_flat.reshape(B, NSLOT, H, H)
